# baseline (speedup 1.0000x reference)
import functools

import jax
import jax.numpy as jnp
from jax import lax
from jax.experimental import pallas as pl
from jax.experimental.pallas import tpu as pltpu
from jax.experimental.pallas import tpu_sc as plsc

_CHUNK = 128
_LANES = 16


def _vsplat(v, lane):
    return lax.gather(
        v, jnp.full((_LANES, 1), lane, jnp.int32),
        lax.GatherDimensionNumbers(
            offset_dims=(), collapsed_slice_dims=(0,), start_index_map=(0,)),
        slice_sizes=(1,), mode=lax.GatherScatterMode.PROMISE_IN_BOUNDS)


def _sc_motor_kernel(n_motor, n_conn, batch, n_neurons, b_lo, n_actions):
    info = plsc.get_sparse_core_info()
    nc, ns = info.num_cores, info.num_subcores
    nw = nc * ns
    npw = n_motor // nw
    rpw = npw * n_conn
    epw = rpw * batch
    n_chunks = epw // _CHUNK
    ppc = _CHUNK // batch
    cpr = n_conn // ppc
    assert n_motor % nw == 0 and epw % _CHUNK == 0 and batch == _LANES
    assert n_conn % _LANES == 0 and npw <= _LANES

    mesh = plsc.VectorSubcoreMesh(core_axis_name="c", subcore_axis_name="s")

    @functools.partial(
        pl.kernel,
        out_type=jax.ShapeDtypeStruct((2, n_actions, batch), jnp.float32),
        mesh=mesh,
        compiler_params=pltpu.CompilerParams(use_tc_tiling_on_sc=False),
        scratch_types=[
            pltpu.VMEM((npw, n_conn), jnp.int32),
            pltpu.VMEM((npw, n_conn), jnp.float32),
            pltpu.VMEM((_LANES,), jnp.float32),
            pltpu.VMEM((epw,), jnp.int32),
            pltpu.VMEM((epw,), jnp.float32),
            pltpu.VMEM((npw, batch), jnp.float32),
            pltpu.VMEM((n_actions, _LANES), jnp.float32),
            pltpu.VMEM((n_actions, batch), jnp.float32),
            pltpu.VMEM((n_actions,), jnp.int32),
            pltpu.VMEM_SHARED((n_actions, batch), jnp.float32),
            pltpu.SemaphoreType.DMA,
            pltpu.SemaphoreType.DMA,
        ],
    )
    def k(idx_hbm, w_hbm, b_hbm, wq_hbm, xf_hbm, out_hbm,
          idx_l, w_l, b_l, idx_v, elems_v, out_v,
          wq_l, qpart, idxrow, qsh, sem, sem2):
        sid = lax.axis_index("s")
        core = lax.axis_index("c")
        wid = sid * nc + core
        pltpu.sync_copy(idx_hbm.at[pl.ds(wid * npw, npw)], idx_l)
        cp_w = pltpu.async_copy(w_hbm.at[pl.ds(wid * npw, npw)], w_l, sem2)
        cp_b = pltpu.async_copy(
            b_hbm.at[pl.ds(b_lo + wid * npw, npw)],
            b_l.at[pl.ds(0, npw)], sem2)
        cp_q = pltpu.async_copy(
            wq_hbm.at[:, pl.ds(wid * npw, npw)],
            wq_l.at[:, pl.ds(0, npw)], sem2)
        iot = lax.iota(jnp.int32, _LANES)
        idxrow[pl.ds(0, _LANES)] = iot
        idxrow[pl.ds(n_actions - _LANES, _LANES)] = iot + (n_actions - _LANES)

        qsz = epw // 4
        offs = lax.iota(jnp.int32, _LANES) * n_neurons
        for p in range(rpw):
            iv = idx_l[p // n_conn,
                       pl.ds((p % n_conn) // _LANES * _LANES, _LANES)]
            flat = _vsplat(iv, p % _LANES) + offs
            idx_v[pl.ds(p * batch, batch)] = flat
            if (p + 1) * batch % qsz == 0:
                q = (p + 1) * batch // qsz - 1
                pltpu.async_copy(
                    xf_hbm.at[idx_v.at[pl.ds(q * qsz, qsz)]],
                    elems_v.at[pl.ds(q * qsz, qsz)], sem)

        cp_w.wait()
        cp_b.wait()
        cp_q.wait()
        pltpu.make_async_copy(xf_hbm.at[pl.ds(0, epw)], elems_v, sem).wait()

        bv = b_l[...]
        for n in range(npw):
            acc = _vsplat(bv, n)
            for h in range(n_conn // _LANES):
                wv = w_l[n, pl.ds(h * _LANES, _LANES)]
                for j in range(_LANES):
                    p = n * n_conn + h * _LANES + j
                    acc = acc + (elems_v[pl.ds(p * batch, batch)]
                                 * _vsplat(wv, j))
            out_v[n, :] = 1.0 / (1.0 + jnp.exp(-acc))

        for a in range(n_actions):
            wv = wq_l[a, :]
            qa = out_v[0, :] * _vsplat(wv, 0)
            for n in range(1, npw):
                qa = qa + out_v[n, :] * _vsplat(wv, n)
            qpart[a, :] = qa
        @pl.when(sid == 0)
        def _():
            pltpu.sync_copy(qpart, qsh)
        plsc.subcore_barrier()
        @pl.when(sid != 0)
        def _():
            pltpu.sync_copy(qpart, qsh.at[idxrow], add=True)
        plsc.subcore_barrier()
        @pl.when(sid == 0)
        def _():
            pltpu.sync_copy(qsh, out_hbm.at[core])

    return k


def kernel(x, W_think, b_think, idx, W_q, b_q):
    batch, n_neurons = x.shape
    n_actions, n_motor = W_q.shape
    out_f, n_conn = idx.shape

    lo = out_f - n_motor
    idx_m = idx[lo:]
    w_m = W_think[lo:]
    xf = x.reshape(-1)

    qp = _sc_motor_kernel(n_motor, n_conn, batch, n_neurons, lo, n_actions)(
        idx_m, w_m, b_think, W_q, xf)
    return (qp[0] + qp[1]).T + b_q


# speedup vs baseline: 1.0181x; 1.0181x over previous
import functools

import jax
import jax.numpy as jnp
from jax import lax
from jax.experimental import pallas as pl
from jax.experimental.pallas import tpu as pltpu
from jax.experimental.pallas import tpu_sc as plsc

_LANES = 16


def _vsplat(v, lane):
    return lax.gather(
        v, jnp.full((_LANES, 1), lane, jnp.int32),
        lax.GatherDimensionNumbers(
            offset_dims=(), collapsed_slice_dims=(0,), start_index_map=(0,)),
        slice_sizes=(1,), mode=lax.GatherScatterMode.PROMISE_IN_BOUNDS)


def _sc_motor_kernel(n_motor, n_conn, batch, n_neurons, b_lo):
    info = plsc.get_sparse_core_info()
    nc, ns = info.num_cores, info.num_subcores
    nw = nc * ns
    npw = n_motor // nw
    rpw = npw * n_conn
    epw = rpw * batch
    assert n_motor % nw == 0 and epw % 4 == 0 and batch == _LANES
    assert n_conn % _LANES == 0 and npw <= _LANES

    mesh = plsc.VectorSubcoreMesh(core_axis_name="c", subcore_axis_name="s")

    @functools.partial(
        pl.kernel,
        out_type=jax.ShapeDtypeStruct((n_motor, batch), jnp.float32),
        mesh=mesh,
        compiler_params=pltpu.CompilerParams(use_tc_tiling_on_sc=False),
        scratch_types=[
            pltpu.VMEM((npw, n_conn), jnp.int32),
            pltpu.VMEM((npw, n_conn), jnp.float32),
            pltpu.VMEM((_LANES,), jnp.float32),
            pltpu.VMEM((epw,), jnp.int32),
            pltpu.VMEM((epw,), jnp.float32),
            pltpu.VMEM((npw, batch), jnp.float32),
            pltpu.SemaphoreType.DMA,
            pltpu.SemaphoreType.DMA,
        ],
    )
    def k(idx_hbm, w_hbm, b_hbm, xf_hbm, out_hbm,
          idx_l, w_l, b_l, idx_v, elems_v, out_v, sem, sem2):
        wid = lax.axis_index("s") * nc + lax.axis_index("c")
        pltpu.sync_copy(idx_hbm.at[pl.ds(wid * npw, npw)], idx_l)
        cp_w = pltpu.async_copy(w_hbm.at[pl.ds(wid * npw, npw)], w_l, sem2)
        cp_b = pltpu.async_copy(
            b_hbm.at[pl.ds(b_lo + wid * npw, npw)],
            b_l.at[pl.ds(0, npw)], sem2)

        qsz = epw // 4
        offs = lax.iota(jnp.int32, _LANES) * n_neurons
        for p in range(rpw):
            iv = idx_l[p // n_conn,
                       pl.ds((p % n_conn) // _LANES * _LANES, _LANES)]
            flat = _vsplat(iv, p % _LANES) + offs
            idx_v[pl.ds(p * batch, batch)] = flat
            if (p + 1) * batch % qsz == 0:
                q = (p + 1) * batch // qsz - 1
                pltpu.async_copy(
                    xf_hbm.at[idx_v.at[pl.ds(q * qsz, qsz)]],
                    elems_v.at[pl.ds(q * qsz, qsz)], sem)

        cp_w.wait()
        cp_b.wait()
        pltpu.make_async_copy(xf_hbm.at[pl.ds(0, epw)], elems_v, sem).wait()

        bv = b_l[...]
        for n in range(npw):
            acc = _vsplat(bv, n)
            for h in range(n_conn // _LANES):
                wv = w_l[n, pl.ds(h * _LANES, _LANES)]
                for j in range(_LANES):
                    p = n * n_conn + h * _LANES + j
                    acc = acc + (elems_v[pl.ds(p * batch, batch)]
                                 * _vsplat(wv, j))
            out_v[n, :] = 1.0 / (1.0 + jnp.exp(-acc))
        pltpu.sync_copy(out_v, out_hbm.at[pl.ds(wid * npw, npw)])

    return k


def _q_head(m_ref, wq_ref, bq_ref, o_ref):
    q = lax.dot_general(
        m_ref[...], wq_ref[...],
        dimension_numbers=(((0,), (1,)), ((), ())),
        preferred_element_type=jnp.float32,
    )
    o_ref[...] = q + bq_ref[...][None, :]


def kernel(x, W_think, b_think, idx, W_q, b_q):
    batch, n_neurons = x.shape
    n_actions, n_motor = W_q.shape
    out_f, n_conn = idx.shape

    lo = out_f - n_motor
    idx_m = idx[lo:]
    w_m = W_think[lo:]
    xf = x.reshape(-1)

    motor_f = _sc_motor_kernel(n_motor, n_conn, batch, n_neurons, lo)(
        idx_m, w_m, b_think, xf)

    q = pl.pallas_call(
        _q_head,
        out_shape=jax.ShapeDtypeStruct((batch, n_actions), jnp.float32),
    )(motor_f, W_q, b_q)
    return q
